# baseline (speedup 1.0000x reference)
import functools

import jax
import jax.numpy as jnp
from jax import lax
from jax.experimental import pallas as pl
from jax.experimental.pallas import tpu as pltpu
from jax.experimental.pallas import tpu_sc as plsc

NUM_NODES = 10000
NUM_EDGES = 320000
HIDDEN = 128

_NC = 2
_NS = 16
_NW = _NC * _NS

_CHUNK = 128
_NCH = 158
_PER_W = _NCH * _CHUNK
_B_PAD = _PER_W * _NS
_Q = 4


def _gather_body(table_hbm, srcs_hbm, dsts_hbm, out_src_hbm, out_dst_hbm,
                 idx_s, idx_d, rs0, rs1, rd0, rd1,
                 gs0, gs1, gd0, gd1, ss0, ss1, sd0, sd1):
    wid = lax.axis_index("s")
    cid = lax.axis_index("c")
    base = wid * _PER_W

    rows = {"s": [rs0, rs1], "d": [rd0, rd1]}
    gsem = {"s": [gs0, gs1], "d": [gd0, gd1]}
    ssem = {"s": [ss0, ss1], "d": [sd0, sd1]}
    idx = {"s": idx_s, "d": idx_d}
    out = {"s": out_src_hbm, "d": out_dst_hbm}

    def ga(x, p, c):
        return pltpu.make_async_copy(
            table_hbm.at[idx[x].at[c]], rows[x][p], gsem[x][p])

    def st(x, p, c):
        return pltpu.make_async_copy(
            rows[x][p], out[x].at[pl.ds(base + c * _CHUNK, _CHUNK)],
            ssem[x][p])

    @pl.when(cid == 0)
    def _():
        pltpu.sync_copy(srcs_hbm.at[wid], idx_s)
        pltpu.sync_copy(dsts_hbm.at[wid], idx_d)

        ga("s", 0, 0).start()
        ga("d", 0, 0).start()
        for c in range(_NCH):
            p = c & 1
            ga("s", p, c).wait()
            ga("d", p, c).wait()
            st("s", p, c).start()
            st("d", p, c).start()
            if c + 1 < _NCH:
                if c >= 1:
                    st("s", 1 - p, c - 1).wait()
                    st("d", 1 - p, c - 1).wait()
                ga("s", 1 - p, c + 1).start()
                ga("d", 1 - p, c + 1).start()
        for x in ("s", "d"):
            st(x, (_NCH - 2) & 1, _NCH - 2).wait()
            st(x, (_NCH - 1) & 1, _NCH - 1).wait()


_sc_gather = functools.partial(
    pl.kernel,
    mesh=plsc.VectorSubcoreMesh(core_axis_name="c", subcore_axis_name="s"),
    out_type=[
        jax.ShapeDtypeStruct((_B_PAD, HIDDEN), jnp.float32),
        jax.ShapeDtypeStruct((_B_PAD, HIDDEN), jnp.float32),
    ],
    scratch_types=(
        [pltpu.VMEM((_NCH, _CHUNK), jnp.int32)] * 2
        + [pltpu.VMEM((_CHUNK, HIDDEN), jnp.float32)] * 4
        + [pltpu.SemaphoreType.DMA] * 8
    ),
)(_gather_body)


_BLK = 3200


def _mm_body(src_ref, dst_ref, dist_ref, ws_ref, wd_ref, we_ref, b_ref, o_ref):
    acc = jnp.dot(src_ref[...], ws_ref[...], preferred_element_type=jnp.float32)
    acc += jnp.dot(dst_ref[...], wd_ref[...], preferred_element_type=jnp.float32)
    acc += jnp.dot(dist_ref[...], we_ref[...], preferred_element_type=jnp.float32)
    o_ref[...] = jnp.maximum(acc + b_ref[...], 0.0)


def kernel(node_edge_feat, dist_feat, srcs, dsts, W, b):
    pad = _B_PAD - NUM_EDGES
    srcs_p = jnp.concatenate([srcs, jnp.zeros((pad,), jnp.int32)])
    dsts_p = jnp.concatenate([dsts, jnp.zeros((pad,), jnp.int32)])
    srcs2d = srcs_p.reshape(_NS, _NCH, _CHUNK)
    dsts2d = dsts_p.reshape(_NS, _NCH, _CHUNK)

    src_g, dst_g = _sc_gather(node_edge_feat, srcs2d, dsts2d)

    ws_t = W[:, :HIDDEN].T
    wd_t = W[:, HIDDEN:2 * HIDDEN].T
    we_t = W[:, 2 * HIDDEN:].T
    b2 = b.reshape(1, HIDDEN)

    feat_spec = pl.BlockSpec((_BLK, HIDDEN), lambda i: (i, 0))
    w_spec = pl.BlockSpec((HIDDEN, HIDDEN), lambda i: (0, 0))
    out = pl.pallas_call(
        _mm_body,
        grid=(NUM_EDGES // _BLK,),
        in_specs=[feat_spec, feat_spec, feat_spec, w_spec, w_spec, w_spec,
                  pl.BlockSpec((1, HIDDEN), lambda i: (0, 0))],
        out_specs=feat_spec,
        out_shape=jax.ShapeDtypeStruct((NUM_EDGES, HIDDEN), jnp.float32),
    )(src_g, dst_g, dist_feat, ws_t, wd_t, we_t, b2)
    return out


# speedup vs baseline: 1.2447x; 1.2447x over previous
import functools

import jax
import jax.numpy as jnp
from jax import lax
from jax.experimental import pallas as pl
from jax.experimental.pallas import tpu as pltpu
from jax.experimental.pallas import tpu_sc as plsc

NUM_NODES = 10000
NUM_EDGES = 320000
HIDDEN = 128

_NC = 2
_NS = 16
_NW = _NC * _NS

_CHUNK = 128
_NCH = 79
_PER_W = _NCH * _CHUNK
_B_PAD = _PER_W * _NW


def _gather_body(table_hbm, srcs_hbm, dsts_hbm, out_src_hbm, out_dst_hbm,
                 idx_s, idx_d, rs0, rs1, rs2, rd0, rd1, rd2,
                 gs0, gs1, gs2, gd0, gd1, gd2,
                 ss0, ss1, ss2, sd0, sd1, sd2):
    wid = lax.axis_index("s") * _NC + lax.axis_index("c")
    base = wid * _PER_W

    rows = {"s": [rs0, rs1, rs2], "d": [rd0, rd1, rd2]}
    gsem = {"s": [gs0, gs1, gs2], "d": [gd0, gd1, gd2]}
    ssem = {"s": [ss0, ss1, ss2], "d": [sd0, sd1, sd2]}
    idx = {"s": idx_s, "d": idx_d}
    out = {"s": out_src_hbm, "d": out_dst_hbm}

    pltpu.sync_copy(srcs_hbm.at[wid], idx_s)
    pltpu.sync_copy(dsts_hbm.at[wid], idx_d)

    def ga(x, p, c):
        return pltpu.make_async_copy(
            table_hbm.at[idx[x].at[c]], rows[x][p], gsem[x][p])

    def st(x, p, c):
        return pltpu.make_async_copy(
            rows[x][p], out[x].at[pl.ds(base + c * _CHUNK, _CHUNK)],
            ssem[x][p])

    ga("s", 0, 0).start()
    ga("d", 0, 0).start()
    ga("s", 1, 1).start()
    ga("d", 1, 1).start()
    for c in range(_NCH):
        p = c % 3
        ga("s", p, c).wait()
        ga("d", p, c).wait()
        st("s", p, c).start()
        st("d", p, c).start()
        if c + 2 < _NCH:
            if c >= 1:
                st("s", (c - 1) % 3, c - 1).wait()
                st("d", (c - 1) % 3, c - 1).wait()
            ga("s", (c + 2) % 3, c + 2).start()
            ga("d", (c + 2) % 3, c + 2).start()
    for x in ("s", "d"):
        st(x, (_NCH - 2) % 3, _NCH - 2).wait()
        st(x, (_NCH - 1) % 3, _NCH - 1).wait()


_sc_gather = functools.partial(
    pl.kernel,
    mesh=plsc.VectorSubcoreMesh(core_axis_name="c", subcore_axis_name="s"),
    out_type=[
        jax.ShapeDtypeStruct((_B_PAD, HIDDEN), jnp.float32),
        jax.ShapeDtypeStruct((_B_PAD, HIDDEN), jnp.float32),
    ],
    scratch_types=(
        [pltpu.VMEM((_NCH, _CHUNK), jnp.int32)] * 2
        + [pltpu.VMEM((_CHUNK, HIDDEN), jnp.float32)] * 6
        + [pltpu.SemaphoreType.DMA] * 12
    ),
)(_gather_body)


_BLK = 6400


def _mm_body(src_ref, dst_ref, dist_ref, ws_ref, wd_ref, we_ref, b_ref, o_ref):
    acc = jnp.dot(src_ref[...], ws_ref[...], preferred_element_type=jnp.float32)
    acc += jnp.dot(dst_ref[...], wd_ref[...], preferred_element_type=jnp.float32)
    acc += jnp.dot(dist_ref[...], we_ref[...], preferred_element_type=jnp.float32)
    o_ref[...] = jnp.maximum(acc + b_ref[...], 0.0)


def kernel(node_edge_feat, dist_feat, srcs, dsts, W, b):
    pad = _B_PAD - NUM_EDGES
    srcs_p = jnp.concatenate([srcs, jnp.zeros((pad,), jnp.int32)])
    dsts_p = jnp.concatenate([dsts, jnp.zeros((pad,), jnp.int32)])
    srcs2d = srcs_p.reshape(_NW, _NCH, _CHUNK)
    dsts2d = dsts_p.reshape(_NW, _NCH, _CHUNK)

    src_g, dst_g = _sc_gather(node_edge_feat, srcs2d, dsts2d)

    ws_t = W[:, :HIDDEN].T
    wd_t = W[:, HIDDEN:2 * HIDDEN].T
    we_t = W[:, 2 * HIDDEN:].T
    b2 = b.reshape(1, HIDDEN)

    feat_spec = pl.BlockSpec((_BLK, HIDDEN), lambda i: (i, 0))
    w_spec = pl.BlockSpec((HIDDEN, HIDDEN), lambda i: (0, 0))
    out = pl.pallas_call(
        _mm_body,
        grid=(NUM_EDGES // _BLK,),
        in_specs=[feat_spec, feat_spec, feat_spec, w_spec, w_spec, w_spec,
                  pl.BlockSpec((1, HIDDEN), lambda i: (0, 0))],
        out_specs=feat_spec,
        out_shape=jax.ShapeDtypeStruct((NUM_EDGES, HIDDEN), jnp.float32),
    )(src_g, dst_g, dist_feat, ws_t, wd_t, we_t, b2)
    return out


# speedup vs baseline: 1.2461x; 1.0011x over previous
import functools

import jax
import jax.numpy as jnp
from jax import lax
from jax.experimental import pallas as pl
from jax.experimental.pallas import tpu as pltpu
from jax.experimental.pallas import tpu_sc as plsc

NUM_NODES = 10000
NUM_EDGES = 320000
HIDDEN = 128

_NC = 2
_NS = 16
_NW = _NC * _NS

_CHUNK = 128
_NCH = 79
_PER_W = _NCH * _CHUNK
_B_PAD = _PER_W * _NW


def _gather_body(table_hbm, srcs_hbm, dsts_hbm, out_src_hbm, out_dst_hbm,
                 idx_s, idx_d, rs0, rs1, rs2, rd0, rd1, rd2,
                 gs0, gs1, gs2, gd0, gd1, gd2,
                 ss0, ss1, ss2, sd0, sd1, sd2):
    wid = lax.axis_index("s") * _NC + lax.axis_index("c")
    base = wid * _PER_W

    rows = {"s": [rs0, rs1, rs2], "d": [rd0, rd1, rd2]}
    gsem = {"s": [gs0, gs1, gs2], "d": [gd0, gd1, gd2]}
    ssem = {"s": [ss0, ss1, ss2], "d": [sd0, sd1, sd2]}
    idx = {"s": idx_s, "d": idx_d}
    out = {"s": out_src_hbm, "d": out_dst_hbm}

    pltpu.sync_copy(srcs_hbm.at[wid], idx_s)
    pltpu.sync_copy(dsts_hbm.at[wid], idx_d)

    def ga(x, p, c):
        return pltpu.make_async_copy(
            table_hbm.at[idx[x].at[c]], rows[x][p], gsem[x][p])

    def st(x, p, c):
        return pltpu.make_async_copy(
            rows[x][p], out[x].at[pl.ds(base + c * _CHUNK, _CHUNK)],
            ssem[x][p])

    ga("s", 0, 0).start()
    ga("d", 0, 0).start()
    ga("s", 1, 1).start()
    ga("d", 1, 1).start()
    for c in range(_NCH):
        p = c % 3
        ga("s", p, c).wait()
        ga("d", p, c).wait()
        st("s", p, c).start()
        st("d", p, c).start()
        if c + 2 < _NCH:
            if c >= 1:
                st("s", (c - 1) % 3, c - 1).wait()
                st("d", (c - 1) % 3, c - 1).wait()
            ga("s", (c + 2) % 3, c + 2).start()
            ga("d", (c + 2) % 3, c + 2).start()
    for x in ("s", "d"):
        st(x, (_NCH - 2) % 3, _NCH - 2).wait()
        st(x, (_NCH - 1) % 3, _NCH - 1).wait()


_sc_gather = functools.partial(
    pl.kernel,
    mesh=plsc.VectorSubcoreMesh(core_axis_name="c", subcore_axis_name="s"),
    out_type=[
        jax.ShapeDtypeStruct((_B_PAD, HIDDEN), jnp.float32),
        jax.ShapeDtypeStruct((_B_PAD, HIDDEN), jnp.float32),
    ],
    scratch_types=(
        [pltpu.VMEM((_NCH, _CHUNK), jnp.int32)] * 2
        + [pltpu.VMEM((_CHUNK, HIDDEN), jnp.float32)] * 6
        + [pltpu.SemaphoreType.DMA] * 12
    ),
)(_gather_body)


_BLK = 8000


def _mm_body(src_ref, dst_ref, dist_ref, ws_ref, wd_ref, we_ref, b_ref, o_ref):
    acc = jnp.dot(src_ref[...], ws_ref[...], preferred_element_type=jnp.float32)
    acc += jnp.dot(dst_ref[...], wd_ref[...], preferred_element_type=jnp.float32)
    acc += jnp.dot(dist_ref[...], we_ref[...], preferred_element_type=jnp.float32)
    o_ref[...] = jnp.maximum(acc + b_ref[...], 0.0)


def kernel(node_edge_feat, dist_feat, srcs, dsts, W, b):
    pad = _B_PAD - NUM_EDGES
    srcs_p = jnp.concatenate([srcs, jnp.zeros((pad,), jnp.int32)])
    dsts_p = jnp.concatenate([dsts, jnp.zeros((pad,), jnp.int32)])
    srcs2d = srcs_p.reshape(_NW, _NCH, _CHUNK)
    dsts2d = dsts_p.reshape(_NW, _NCH, _CHUNK)

    src_g, dst_g = _sc_gather(node_edge_feat, srcs2d, dsts2d)

    ws_t = W[:, :HIDDEN].T
    wd_t = W[:, HIDDEN:2 * HIDDEN].T
    we_t = W[:, 2 * HIDDEN:].T
    b2 = b.reshape(1, HIDDEN)

    feat_spec = pl.BlockSpec((_BLK, HIDDEN), lambda i: (i, 0))
    w_spec = pl.BlockSpec((HIDDEN, HIDDEN), lambda i: (0, 0))
    out = pl.pallas_call(
        _mm_body,
        grid=(NUM_EDGES // _BLK,),
        in_specs=[feat_spec, feat_spec, feat_spec, w_spec, w_spec, w_spec,
                  pl.BlockSpec((1, HIDDEN), lambda i: (0, 0))],
        out_specs=feat_spec,
        out_shape=jax.ShapeDtypeStruct((NUM_EDGES, HIDDEN), jnp.float32),
    )(src_g, dst_g, dist_feat, ws_t, wd_t, we_t, b2)
    return out
